# baseline (speedup 1.0000x reference)
import jax
import jax.numpy as jnp
from jax import lax
from jax.experimental import pallas as pl
from jax.experimental.pallas import tpu as pltpu
from jax.experimental.pallas import tpu_sc as plsc

N = 10000
E = 320000
D = 128
H = 16
OUT = 128
G = 16

NC = 2
NS = 16
NW = NC * NS
CPB = 1024
NBLK = 10
EPB = NW * NBLK
EPAD = EPB * CPB
NPAD = 10240
STRIPE = NPAD // NS
DUMMY_DST = NPAD - 16
VL = 16


CPB2 = 512
NBLK2 = 20


def _sc_mesh():
    return plsc.VectorSubcoreMesh(core_axis_name="c", subcore_axis_name="s",
                                  num_cores=NC, num_subcores=NS)


def _edge_scatter_ring(y_sp, acc, src_v, dst_v, rows, gsems, ssems, nblk):
    pltpu.async_copy(y_sp.at[src_v.at[0]], rows[0], gsems[0])
    pltpu.async_copy(y_sp.at[src_v.at[1]], rows[1], gsems[1])
    for k in range(nblk):
        b = k % 4
        pltpu.make_async_copy(y_sp.at[src_v.at[k]], rows[b], gsems[b]).wait()
        pltpu.async_copy(rows[b], acc.at[dst_v.at[k]], ssems[b], add=True)
        nxt = k + 2
        if nxt < nblk:
            nb = nxt % 4
            if nxt >= 4:
                pltpu.make_async_copy(rows[nb], acc.at[dst_v.at[nxt - 4]],
                                      ssems[nb]).wait()
            pltpu.async_copy(y_sp.at[src_v.at[nxt]], rows[nb], gsems[nb])
    for k in range(max(0, nblk - 4), nblk):
        b = k % 4
        pltpu.make_async_copy(rows[b], acc.at[dst_v.at[k]], ssems[b]).wait()


DEG_NB = E // CPB2
DEG_HI = DEG_NB - NW * (DEG_NB // NW)
DEG_LO = DEG_NB // NW


def _deg_body(ei_hbm, out_hbm, dst_v, ones_v, stripe_v, acc, isem, ssem):
    c = lax.axis_index("c")
    s = lax.axis_index("s")
    wid = s * NC + c
    row0 = s * STRIPE
    base = DEG_LO * wid + jnp.minimum(wid, DEG_HI)
    pltpu.async_copy(ei_hbm.at[1, pl.ds(base, DEG_LO)],
                     dst_v.at[pl.ds(0, DEG_LO)], isem)

    @pl.when(wid < DEG_HI)
    def _():
        pltpu.async_copy(ei_hbm.at[1, pl.ds(base + DEG_LO, 1)],
                         dst_v.at[pl.ds(DEG_LO, 1)], isem)

    for i in range(CPB2 // VL):
        ones_v[pl.ds(i * VL, VL)] = jnp.ones((VL,), jnp.float32)
    for i in range(STRIPE // VL):
        stripe_v[pl.ds(i * VL, VL)] = jnp.ones((VL,), jnp.float32)
    pltpu.sync_copy(stripe_v, acc.at[pl.ds(row0, STRIPE)])
    pltpu.make_async_copy(ei_hbm.at[1, pl.ds(base, DEG_LO)],
                          dst_v.at[pl.ds(0, DEG_LO)], isem).wait()

    @pl.when(wid < DEG_HI)
    def _():
        pltpu.make_async_copy(ei_hbm.at[1, pl.ds(base + DEG_LO, 1)],
                              dst_v.at[pl.ds(DEG_LO, 1)], isem).wait()

    plsc.subcore_barrier()

    for g in range(DEG_LO):
        pltpu.async_copy(ones_v, acc.at[dst_v.at[g]], ssem, add=True)

    @pl.when(wid < DEG_HI)
    def _():
        pltpu.async_copy(ones_v, acc.at[dst_v.at[DEG_LO]], ssem, add=True)

    for g in range(DEG_LO):
        pltpu.make_async_copy(ones_v, acc.at[dst_v.at[g]], ssem).wait()

    @pl.when(wid < DEG_HI)
    def _():
        pltpu.make_async_copy(ones_v, acc.at[dst_v.at[DEG_LO]], ssem).wait()

    plsc.subcore_barrier()
    pltpu.sync_copy(acc.at[pl.ds(row0, STRIPE)], stripe_v)
    pltpu.sync_copy(stripe_v, out_hbm.at[c, pl.ds(row0, STRIPE)])


def _sc_degree(ei_raw):
    kern = pl.kernel(
        _deg_body,
        out_type=jax.ShapeDtypeStruct((NC, NPAD), jnp.float32),
        mesh=_sc_mesh(),
        scratch_types=[
            pltpu.VMEM((DEG_LO + 1, CPB2), jnp.int32),
            pltpu.VMEM((CPB2,), jnp.float32),
            pltpu.VMEM((STRIPE,), jnp.float32),
            pltpu.VMEM_SHARED((NPAD,), jnp.float32),
            pltpu.SemaphoreType.DMA,
            pltpu.SemaphoreType.DMA,
        ],
        compiler_params=pltpu.CompilerParams(use_tc_tiling_on_sc=False),
    )
    return kern(ei_raw)


def _fused1_body(degp_hbm, hw_hbm, ei_hbm, p_hbm, y1_hbm, dinv_hbm,
                 src_v, dst_v, rows_a, rows_b, rows_c, rows_d,
                 hwv, y1v, dv0, dv1, dinv_v, y_sp, acc,
                 ga, gb, gc, gd, sa, sb, sc, sd):
    c = lax.axis_index("c")
    s = lax.axis_index("s")
    wid = s * NC + c
    row0 = s * STRIPE
    cc = lax.broadcasted_iota(jnp.int32, (16,), 0)

    pltpu.sync_copy(degp_hbm.at[0, pl.ds(row0, STRIPE)], dv0)
    pltpu.sync_copy(degp_hbm.at[1, pl.ds(row0, STRIPE)], dv1)
    pltpu.sync_copy(hw_hbm.at[pl.ds(row0, STRIPE)], hwv)
    pltpu.sync_copy(ei_hbm.at[0, pl.ds(wid * NBLK, NBLK)], src_v)
    pltpu.sync_copy(ei_hbm.at[1, pl.ds(wid * NBLK, NBLK)], dst_v)

    def blk(i, carry):
        x = dv0[pl.ds(i * 16, 16)] + dv1[pl.ds(i * 16, 16)] - 1.0
        ii = plsc.bitcast(x, jnp.int32)
        yy = plsc.bitcast(jnp.int32(0x5F3759DF) - (ii >> 1), jnp.float32)
        for _ in range(3):
            yy = yy * (1.5 - 0.5 * x * yy * yy)
        dinv_v[pl.ds(i * 16, 16)] = yy
        for j in range(16):
            r = i * 16 + j
            rr = jnp.full((16,), r, jnp.int32)
            row = plsc.load_gather(hwv, [rr, cc]) * yy[j]
            plsc.store_scatter(y1v, [rr, cc], row)
        return carry

    lax.fori_loop(0, STRIPE // 16, blk, 0)
    pltpu.sync_copy(y1v, y_sp.at[pl.ds(row0, STRIPE)])
    pltpu.sync_copy(y1v, acc.at[pl.ds(row0, STRIPE)])

    @pl.when(c == 0)
    def _():
        pltpu.sync_copy(y1v, y1_hbm.at[pl.ds(row0, STRIPE)])
        pltpu.sync_copy(dinv_v, dinv_hbm.at[pl.ds(row0, STRIPE)])

    plsc.subcore_barrier()

    _edge_scatter_ring(y_sp, acc, src_v, dst_v,
                       (rows_a, rows_b, rows_c, rows_d),
                       (ga, gb, gc, gd), (sa, sb, sc, sd), NBLK)
    plsc.subcore_barrier()
    pltpu.sync_copy(acc.at[pl.ds(row0, STRIPE)], hwv)
    pltpu.sync_copy(hwv, p_hbm.at[c, pl.ds(row0, STRIPE)])


def _sc_fused1(degp, hw_pad, ei3):
    kern = pl.kernel(
        _fused1_body,
        out_type=(
            jax.ShapeDtypeStruct((NC, NPAD, H), jnp.float32),
            jax.ShapeDtypeStruct((NPAD, H), jnp.float32),
            jax.ShapeDtypeStruct((NPAD,), jnp.float32),
        ),
        mesh=_sc_mesh(),
        scratch_types=[
            pltpu.VMEM((NBLK, CPB), jnp.int32),
            pltpu.VMEM((NBLK, CPB), jnp.int32),
            pltpu.VMEM((CPB, H), jnp.float32),
            pltpu.VMEM((CPB, H), jnp.float32),
            pltpu.VMEM((CPB, H), jnp.float32),
            pltpu.VMEM((CPB, H), jnp.float32),
            pltpu.VMEM((STRIPE, H), jnp.float32),
            pltpu.VMEM((STRIPE, H), jnp.float32),
            pltpu.VMEM((STRIPE,), jnp.float32),
            pltpu.VMEM((STRIPE,), jnp.float32),
            pltpu.VMEM((STRIPE,), jnp.float32),
            pltpu.VMEM_SHARED((NPAD, H), jnp.float32),
            pltpu.VMEM_SHARED((NPAD, H), jnp.float32),
            pltpu.SemaphoreType.DMA,
            pltpu.SemaphoreType.DMA,
            pltpu.SemaphoreType.DMA,
            pltpu.SemaphoreType.DMA,
            pltpu.SemaphoreType.DMA,
            pltpu.SemaphoreType.DMA,
            pltpu.SemaphoreType.DMA,
            pltpu.SemaphoreType.DMA,
        ],
        compiler_params=pltpu.CompilerParams(use_tc_tiling_on_sc=False,
                                             needs_layout_passes=False),
    )
    return kern(degp, hw_pad, ei3)


def _mm_body(x_ref, w_ref, o_ref):
    hw = jnp.dot(x_ref[...], w_ref[...], preferred_element_type=jnp.float32)
    o_ref[...] = jnp.concatenate(
        [hw, jnp.zeros((NPAD - N, H), jnp.float32)], axis=0)


def _tc_xw(x, W1):
    return pl.pallas_call(
        _mm_body,
        out_shape=jax.ShapeDtypeStruct((NPAD, H), jnp.float32),
    )(x, W1)


def _fused2_body(p_hbm, y1_hbm, dinv_hbm, b1_hbm, batch_hbm, ei_hbm, out_hbm,
                 src_v, dst_v, rows_a, rows_b, rows_c, rows_d,
                 pv0, pv1, y1v, y2v, accv, dinv_v, batch_v, b1_v, pool_v,
                 y_sp, acc, ga, gb, gc, gd, sa, sb, sc, sd):
    c = lax.axis_index("c")
    s = lax.axis_index("s")
    wid = s * NC + c
    row0 = s * STRIPE
    cc = lax.broadcasted_iota(jnp.int32, (16,), 0)

    pltpu.sync_copy(p_hbm.at[0, pl.ds(row0, STRIPE)], pv0)
    pltpu.sync_copy(p_hbm.at[1, pl.ds(row0, STRIPE)], pv1)
    pltpu.sync_copy(y1_hbm.at[pl.ds(row0, STRIPE)], y1v)
    pltpu.sync_copy(dinv_hbm.at[pl.ds(row0, STRIPE)], dinv_v)
    pltpu.sync_copy(batch_hbm.at[pl.ds(row0, STRIPE)], batch_v)
    pltpu.sync_copy(b1_hbm, b1_v)
    pltpu.sync_copy(ei_hbm.at[0, pl.ds(wid * NBLK2, NBLK2)], src_v)
    pltpu.sync_copy(ei_hbm.at[1, pl.ds(wid * NBLK2, NBLK2)], dst_v)
    b1row = b1_v[...]

    def relu_blk(i, carry):
        dvec = dinv_v[pl.ds(i * 16, 16)]
        for j in range(16):
            r = i * 16 + j
            rr = jnp.full((16,), r, jnp.int32)
            z = (plsc.load_gather(pv0, [rr, cc])
                 + plsc.load_gather(pv1, [rr, cc])
                 - plsc.load_gather(y1v, [rr, cc]))
            d = dvec[j]
            h = jnp.maximum(z * d + b1row, 0.0)
            h = jnp.where(row0 + r < N, h, 0.0)
            plsc.store_scatter(y2v, [rr, cc], h * d)
        return carry

    lax.fori_loop(0, STRIPE // 16, relu_blk, 0)
    pltpu.sync_copy(y2v, y_sp.at[pl.ds(row0, STRIPE)])
    pltpu.sync_copy(y2v, acc.at[pl.ds(row0, STRIPE)])
    plsc.subcore_barrier()

    _edge_scatter_ring(y_sp, acc, src_v, dst_v,
                       (rows_a, rows_b, rows_c, rows_d),
                       (ga, gb, gc, gd), (sa, sb, sc, sd), NBLK2)
    plsc.subcore_barrier()

    pltpu.sync_copy(acc.at[pl.ds(row0, STRIPE)], accv)
    for i in range(G + 1):
        pool_v[pl.ds(i * H, H)] = jnp.zeros((H,), jnp.float32)
    sub = jnp.where(c == 0, 1.0, 0.0)

    def pool_blk(i, carry):
        dvec = dinv_v[pl.ds(i * 16, 16)]
        bvec = batch_v[pl.ds(i * 16, 16)]
        for j in range(16):
            r = i * 16 + j
            rr = jnp.full((16,), r, jnp.int32)
            a = (plsc.load_gather(accv, [rr, cc])
                 - sub * plsc.load_gather(y2v, [rr, cc]))
            val = a * dvec[j]
            idx = bvec[j] * H + cc
            plsc.addupdate_scatter(pool_v, [idx], val)
        return carry

    lax.fori_loop(0, STRIPE // 16, pool_blk, 0)
    pltpu.sync_copy(pool_v, out_hbm.at[c, s])


def _sc_fused2(p1, y1, dinv, b1, batch_pad, ei3):
    kern = pl.kernel(
        _fused2_body,
        out_type=jax.ShapeDtypeStruct((NC, NS, (G + 1) * H), jnp.float32),
        mesh=_sc_mesh(),
        scratch_types=[
            pltpu.VMEM((NBLK2, CPB2), jnp.int32),
            pltpu.VMEM((NBLK2, CPB2), jnp.int32),
            pltpu.VMEM((CPB2, H), jnp.float32),
            pltpu.VMEM((CPB2, H), jnp.float32),
            pltpu.VMEM((CPB2, H), jnp.float32),
            pltpu.VMEM((CPB2, H), jnp.float32),
            pltpu.VMEM((STRIPE, H), jnp.float32),
            pltpu.VMEM((STRIPE, H), jnp.float32),
            pltpu.VMEM((STRIPE, H), jnp.float32),
            pltpu.VMEM((STRIPE, H), jnp.float32),
            pltpu.VMEM((STRIPE, H), jnp.float32),
            pltpu.VMEM((STRIPE,), jnp.float32),
            pltpu.VMEM((STRIPE,), jnp.int32),
            pltpu.VMEM((H,), jnp.float32),
            pltpu.VMEM(((G + 1) * H,), jnp.float32),
            pltpu.VMEM_SHARED((NPAD, H), jnp.float32),
            pltpu.VMEM_SHARED((NPAD, H), jnp.float32),
            pltpu.SemaphoreType.DMA,
            pltpu.SemaphoreType.DMA,
            pltpu.SemaphoreType.DMA,
            pltpu.SemaphoreType.DMA,
            pltpu.SemaphoreType.DMA,
            pltpu.SemaphoreType.DMA,
            pltpu.SemaphoreType.DMA,
            pltpu.SemaphoreType.DMA,
        ],
        compiler_params=pltpu.CompilerParams(use_tc_tiling_on_sc=False,
                                             needs_layout_passes=False),
    )
    return kern(p1, y1, dinv, b1, batch_pad, ei3)


def _final_body(parts_ref, batch_ref, w2_ref, b2_ref, o_ref):
    ps = jnp.sum(parts_ref[...], axis=0)
    pool = ps[:G, :]
    gids = lax.broadcasted_iota(jnp.int32, (G, N), 0)
    onehot = (gids == batch_ref[...]).astype(jnp.float32)
    counts = jnp.sum(onehot, axis=1, keepdims=True)
    o_ref[...] = (jnp.dot(pool, w2_ref[...],
                          preferred_element_type=jnp.float32)
                  + counts * b2_ref[...])


def _tc_final(parts, batch_row, W2, b2):
    return pl.pallas_call(
        _final_body,
        out_shape=jax.ShapeDtypeStruct((G, OUT), jnp.float32),
    )(parts, batch_row, W2, b2)


@jax.jit
def _run(x, edge_index, batch, W1, b1, W2, b2):
    pad = jnp.stack([jnp.zeros((EPAD - E,), jnp.int32),
                     jnp.full((EPAD - E,), DUMMY_DST, jnp.int32)])
    ei_pad = jnp.concatenate([edge_index, pad], axis=1)
    ei3 = ei_pad.reshape(2, EPB, CPB)
    ei3b = ei_pad.reshape(2, NW * NBLK2, CPB2)
    ei_raw = edge_index.reshape(2, DEG_NB, CPB2)

    degp = _sc_degree(ei_raw)
    hw_pad = _tc_xw(x, W1)
    p1, y1, dinv = _sc_fused1(degp, hw_pad, ei3)
    batch_pad = jnp.pad(batch, (0, NPAD - N), constant_values=G)
    parts = _sc_fused2(p1, y1, dinv, b1, batch_pad,
                       ei3b)
    parts3 = parts.reshape(NW, G + 1, H)
    out = _tc_final(parts3, batch.reshape(1, N), W2, b2.reshape(1, OUT))
    return out


def kernel(x, edge_index, batch, W1, b1, W2, b2, training=False):
    return _run(x, edge_index, batch, W1, b1, W2, b2)


# speedup vs baseline: 1.0193x; 1.0193x over previous
import jax
import jax.numpy as jnp
from jax import lax
from jax.experimental import pallas as pl
from jax.experimental.pallas import tpu as pltpu
from jax.experimental.pallas import tpu_sc as plsc

N = 10000
E = 320000
D = 128
H = 16
OUT = 128
G = 16

NC = 2
NS = 16
NW = NC * NS
CPB = 1024
NBLK = 10
EPB = NW * NBLK
EPAD = EPB * CPB
NPAD = 10240
STRIPE = NPAD // NS
DUMMY_DST = NPAD - 16
VL = 16


CPB2 = 512
NBLK2 = 20


def _sc_mesh():
    return plsc.VectorSubcoreMesh(core_axis_name="c", subcore_axis_name="s",
                                  num_cores=NC, num_subcores=NS)


def _edge_scatter_ring(y_sp, acc, src_v, dst_v, rows, gsems, ssems, nblk):
    pltpu.async_copy(y_sp.at[src_v.at[0]], rows[0], gsems[0])
    pltpu.async_copy(y_sp.at[src_v.at[1]], rows[1], gsems[1])
    for k in range(nblk):
        b = k % 4
        pltpu.make_async_copy(y_sp.at[src_v.at[k]], rows[b], gsems[b]).wait()
        pltpu.async_copy(rows[b], acc.at[dst_v.at[k]], ssems[b], add=True)
        nxt = k + 2
        if nxt < nblk:
            nb = nxt % 4
            if nxt >= 4:
                pltpu.make_async_copy(rows[nb], acc.at[dst_v.at[nxt - 4]],
                                      ssems[nb]).wait()
            pltpu.async_copy(y_sp.at[src_v.at[nxt]], rows[nb], gsems[nb])
    for k in range(max(0, nblk - 4), nblk):
        b = k % 4
        pltpu.make_async_copy(rows[b], acc.at[dst_v.at[k]], ssems[b]).wait()


def _deg_body(ei_hbm, out_hbm, dst_v, ones_v, stripe_v, acc, isem, ssem):
    c = lax.axis_index("c")
    s = lax.axis_index("s")
    wid = s * NC + c
    row0 = s * STRIPE
    pltpu.async_copy(ei_hbm.at[1, pl.ds(wid * NBLK, NBLK)], dst_v, isem)
    for i in range(CPB // VL):
        ones_v[pl.ds(i * VL, VL)] = jnp.ones((VL,), jnp.float32)
    for i in range(STRIPE // VL):
        stripe_v[pl.ds(i * VL, VL)] = jnp.ones((VL,), jnp.float32)
    pltpu.sync_copy(stripe_v, acc.at[pl.ds(row0, STRIPE)])
    pltpu.make_async_copy(ei_hbm.at[1, pl.ds(wid * NBLK, NBLK)], dst_v,
                          isem).wait()
    plsc.subcore_barrier()

    for g in range(NBLK):
        pltpu.async_copy(ones_v, acc.at[dst_v.at[g]], ssem, add=True)
    for g in range(NBLK):
        pltpu.make_async_copy(ones_v, acc.at[dst_v.at[g]], ssem).wait()
    plsc.subcore_barrier()
    pltpu.sync_copy(acc.at[pl.ds(row0, STRIPE)], stripe_v)
    pltpu.sync_copy(stripe_v, out_hbm.at[c, pl.ds(row0, STRIPE)])


def _sc_degree(ei3):
    kern = pl.kernel(
        _deg_body,
        out_type=jax.ShapeDtypeStruct((NC, NPAD), jnp.float32),
        mesh=_sc_mesh(),
        scratch_types=[
            pltpu.VMEM((NBLK, CPB), jnp.int32),
            pltpu.VMEM((CPB,), jnp.float32),
            pltpu.VMEM((STRIPE,), jnp.float32),
            pltpu.VMEM_SHARED((NPAD,), jnp.float32),
            pltpu.SemaphoreType.DMA,
            pltpu.SemaphoreType.DMA,
        ],
        compiler_params=pltpu.CompilerParams(use_tc_tiling_on_sc=False),
    )
    return kern(ei3)


def _fused1_body(degp_hbm, hw_hbm, ei_hbm, p_hbm, y1_hbm, dinv_hbm,
                 src_v, dst_v, rows_a, rows_b, rows_c, rows_d,
                 hwv, y1v, dv0, dv1, dinv_v, y_sp, acc,
                 ga, gb, gc, gd, sa, sb, sc, sd):
    c = lax.axis_index("c")
    s = lax.axis_index("s")
    wid = s * NC + c
    row0 = s * STRIPE
    cc = lax.broadcasted_iota(jnp.int32, (16,), 0)

    pltpu.sync_copy(degp_hbm.at[0, pl.ds(row0, STRIPE)], dv0)
    pltpu.sync_copy(degp_hbm.at[1, pl.ds(row0, STRIPE)], dv1)
    pltpu.sync_copy(hw_hbm.at[pl.ds(row0, STRIPE)], hwv)
    pltpu.sync_copy(ei_hbm.at[0, pl.ds(wid * NBLK, NBLK)], src_v)
    pltpu.sync_copy(ei_hbm.at[1, pl.ds(wid * NBLK, NBLK)], dst_v)

    def blk(i, carry):
        x = dv0[pl.ds(i * 16, 16)] + dv1[pl.ds(i * 16, 16)] - 1.0
        ii = plsc.bitcast(x, jnp.int32)
        yy = plsc.bitcast(jnp.int32(0x5F3759DF) - (ii >> 1), jnp.float32)
        for _ in range(3):
            yy = yy * (1.5 - 0.5 * x * yy * yy)
        dinv_v[pl.ds(i * 16, 16)] = yy
        for j in range(16):
            r = i * 16 + j
            rr = jnp.full((16,), r, jnp.int32)
            row = plsc.load_gather(hwv, [rr, cc]) * yy[j]
            plsc.store_scatter(y1v, [rr, cc], row)
        return carry

    lax.fori_loop(0, STRIPE // 16, blk, 0)
    pltpu.sync_copy(y1v, y_sp.at[pl.ds(row0, STRIPE)])
    pltpu.sync_copy(y1v, acc.at[pl.ds(row0, STRIPE)])

    @pl.when(c == 0)
    def _():
        pltpu.sync_copy(y1v, y1_hbm.at[pl.ds(row0, STRIPE)])
        pltpu.sync_copy(dinv_v, dinv_hbm.at[pl.ds(row0, STRIPE)])

    plsc.subcore_barrier()

    _edge_scatter_ring(y_sp, acc, src_v, dst_v,
                       (rows_a, rows_b, rows_c, rows_d),
                       (ga, gb, gc, gd), (sa, sb, sc, sd), NBLK)
    plsc.subcore_barrier()
    pltpu.sync_copy(acc.at[pl.ds(row0, STRIPE)], hwv)
    pltpu.sync_copy(hwv, p_hbm.at[c, pl.ds(row0, STRIPE)])


def _sc_fused1(degp, hw_pad, ei3):
    kern = pl.kernel(
        _fused1_body,
        out_type=(
            jax.ShapeDtypeStruct((NC, NPAD, H), jnp.float32),
            jax.ShapeDtypeStruct((NPAD, H), jnp.float32),
            jax.ShapeDtypeStruct((NPAD,), jnp.float32),
        ),
        mesh=_sc_mesh(),
        scratch_types=[
            pltpu.VMEM((NBLK, CPB), jnp.int32),
            pltpu.VMEM((NBLK, CPB), jnp.int32),
            pltpu.VMEM((CPB, H), jnp.float32),
            pltpu.VMEM((CPB, H), jnp.float32),
            pltpu.VMEM((CPB, H), jnp.float32),
            pltpu.VMEM((CPB, H), jnp.float32),
            pltpu.VMEM((STRIPE, H), jnp.float32),
            pltpu.VMEM((STRIPE, H), jnp.float32),
            pltpu.VMEM((STRIPE,), jnp.float32),
            pltpu.VMEM((STRIPE,), jnp.float32),
            pltpu.VMEM((STRIPE,), jnp.float32),
            pltpu.VMEM_SHARED((NPAD, H), jnp.float32),
            pltpu.VMEM_SHARED((NPAD, H), jnp.float32),
            pltpu.SemaphoreType.DMA,
            pltpu.SemaphoreType.DMA,
            pltpu.SemaphoreType.DMA,
            pltpu.SemaphoreType.DMA,
            pltpu.SemaphoreType.DMA,
            pltpu.SemaphoreType.DMA,
            pltpu.SemaphoreType.DMA,
            pltpu.SemaphoreType.DMA,
        ],
        compiler_params=pltpu.CompilerParams(use_tc_tiling_on_sc=False,
                                             needs_layout_passes=False),
    )
    return kern(degp, hw_pad, ei3)


def _mm_body(x_ref, w_ref, o_ref):
    hw = jnp.dot(x_ref[...], w_ref[...], preferred_element_type=jnp.float32)
    o_ref[...] = jnp.concatenate(
        [hw, jnp.zeros((NPAD - N, H), jnp.float32)], axis=0)


def _tc_xw(x, W1):
    return pl.pallas_call(
        _mm_body,
        out_shape=jax.ShapeDtypeStruct((NPAD, H), jnp.float32),
    )(x, W1)


def _fused2_body(p_hbm, y1_hbm, dinv_hbm, b1_hbm, batch_hbm, ei_hbm, out_hbm,
                 src_v, dst_v, rows_a, rows_b, rows_c, rows_d,
                 pv0, pv1, y1v, y2v, accv, dinv_v, batch_v, b1_v, pool_v,
                 y_sp, acc, ga, gb, gc, gd, sa, sb, sc, sd):
    c = lax.axis_index("c")
    s = lax.axis_index("s")
    wid = s * NC + c
    row0 = s * STRIPE
    cc = lax.broadcasted_iota(jnp.int32, (16,), 0)

    pltpu.sync_copy(p_hbm.at[0, pl.ds(row0, STRIPE)], pv0)
    pltpu.sync_copy(p_hbm.at[1, pl.ds(row0, STRIPE)], pv1)
    pltpu.sync_copy(y1_hbm.at[pl.ds(row0, STRIPE)], y1v)
    pltpu.sync_copy(dinv_hbm.at[pl.ds(row0, STRIPE)], dinv_v)
    pltpu.sync_copy(batch_hbm.at[pl.ds(row0, STRIPE)], batch_v)
    pltpu.sync_copy(b1_hbm, b1_v)
    pltpu.sync_copy(ei_hbm.at[0, pl.ds(wid * NBLK2, NBLK2)], src_v)
    pltpu.sync_copy(ei_hbm.at[1, pl.ds(wid * NBLK2, NBLK2)], dst_v)
    b1row = b1_v[...]

    def relu_blk(i, carry):
        dvec = dinv_v[pl.ds(i * 16, 16)]
        for j in range(16):
            r = i * 16 + j
            rr = jnp.full((16,), r, jnp.int32)
            z = (plsc.load_gather(pv0, [rr, cc])
                 + plsc.load_gather(pv1, [rr, cc])
                 - plsc.load_gather(y1v, [rr, cc]))
            d = dvec[j]
            h = jnp.maximum(z * d + b1row, 0.0)
            h = jnp.where(row0 + r < N, h, 0.0)
            plsc.store_scatter(y2v, [rr, cc], h * d)
        return carry

    lax.fori_loop(0, STRIPE // 16, relu_blk, 0)
    pltpu.sync_copy(y2v, y_sp.at[pl.ds(row0, STRIPE)])
    pltpu.sync_copy(y2v, acc.at[pl.ds(row0, STRIPE)])
    plsc.subcore_barrier()

    _edge_scatter_ring(y_sp, acc, src_v, dst_v,
                       (rows_a, rows_b, rows_c, rows_d),
                       (ga, gb, gc, gd), (sa, sb, sc, sd), NBLK2)
    plsc.subcore_barrier()

    pltpu.sync_copy(acc.at[pl.ds(row0, STRIPE)], accv)
    for i in range(G + 1):
        pool_v[pl.ds(i * H, H)] = jnp.zeros((H,), jnp.float32)
    sub = jnp.where(c == 0, 1.0, 0.0)

    def pool_blk(i, carry):
        dvec = dinv_v[pl.ds(i * 16, 16)]
        bvec = batch_v[pl.ds(i * 16, 16)]
        for j in range(16):
            r = i * 16 + j
            rr = jnp.full((16,), r, jnp.int32)
            a = (plsc.load_gather(accv, [rr, cc])
                 - sub * plsc.load_gather(y2v, [rr, cc]))
            val = a * dvec[j]
            idx = bvec[j] * H + cc
            plsc.addupdate_scatter(pool_v, [idx], val)
        return carry

    lax.fori_loop(0, STRIPE // 16, pool_blk, 0)
    pltpu.sync_copy(pool_v, out_hbm.at[c, s])


def _sc_fused2(p1, y1, dinv, b1, batch_pad, ei3):
    kern = pl.kernel(
        _fused2_body,
        out_type=jax.ShapeDtypeStruct((NC, NS, (G + 1) * H), jnp.float32),
        mesh=_sc_mesh(),
        scratch_types=[
            pltpu.VMEM((NBLK2, CPB2), jnp.int32),
            pltpu.VMEM((NBLK2, CPB2), jnp.int32),
            pltpu.VMEM((CPB2, H), jnp.float32),
            pltpu.VMEM((CPB2, H), jnp.float32),
            pltpu.VMEM((CPB2, H), jnp.float32),
            pltpu.VMEM((CPB2, H), jnp.float32),
            pltpu.VMEM((STRIPE, H), jnp.float32),
            pltpu.VMEM((STRIPE, H), jnp.float32),
            pltpu.VMEM((STRIPE, H), jnp.float32),
            pltpu.VMEM((STRIPE, H), jnp.float32),
            pltpu.VMEM((STRIPE, H), jnp.float32),
            pltpu.VMEM((STRIPE,), jnp.float32),
            pltpu.VMEM((STRIPE,), jnp.int32),
            pltpu.VMEM((H,), jnp.float32),
            pltpu.VMEM(((G + 1) * H,), jnp.float32),
            pltpu.VMEM_SHARED((NPAD, H), jnp.float32),
            pltpu.VMEM_SHARED((NPAD, H), jnp.float32),
            pltpu.SemaphoreType.DMA,
            pltpu.SemaphoreType.DMA,
            pltpu.SemaphoreType.DMA,
            pltpu.SemaphoreType.DMA,
            pltpu.SemaphoreType.DMA,
            pltpu.SemaphoreType.DMA,
            pltpu.SemaphoreType.DMA,
            pltpu.SemaphoreType.DMA,
        ],
        compiler_params=pltpu.CompilerParams(use_tc_tiling_on_sc=False,
                                             needs_layout_passes=False),
    )
    return kern(p1, y1, dinv, b1, batch_pad, ei3)


def _final_body(parts_ref, batch_ref, w2_ref, b2_ref, o_ref):
    ps = jnp.sum(parts_ref[...], axis=0)
    pool = ps[:G, :]
    gids = lax.broadcasted_iota(jnp.int32, (G, N), 0)
    onehot = (gids == batch_ref[...]).astype(jnp.float32)
    counts = jnp.sum(onehot, axis=1, keepdims=True)
    o_ref[...] = (jnp.dot(pool, w2_ref[...],
                          preferred_element_type=jnp.float32)
                  + counts * b2_ref[...])


def _tc_final(parts, batch_row, W2, b2):
    return pl.pallas_call(
        _final_body,
        out_shape=jax.ShapeDtypeStruct((G, OUT), jnp.float32),
    )(parts, batch_row, W2, b2)


@jax.jit
def _run(x, edge_index, batch, W1, b1, W2, b2):
    pad = jnp.stack([jnp.zeros((EPAD - E,), jnp.int32),
                     jnp.full((EPAD - E,), DUMMY_DST, jnp.int32)])
    ei_pad = jnp.concatenate([edge_index, pad], axis=1)
    ei3 = ei_pad.reshape(2, EPB, CPB)
    ei3b = ei_pad.reshape(2, NW * NBLK2, CPB2)

    degp = _sc_degree(ei3)
    hw_pad = _tc_xw(x, W1)
    p1, y1, dinv = _sc_fused1(degp, hw_pad, ei3)
    batch_pad = jnp.pad(batch, (0, NPAD - N), constant_values=G)
    parts = _sc_fused2(p1, y1, dinv, b1, batch_pad,
                       ei3b)
    parts3 = parts.reshape(NW, G + 1, H)
    out = _tc_final(parts3, batch.reshape(1, N), W2, b2.reshape(1, OUT))
    return out


def kernel(x, edge_index, batch, W1, b1, W2, b2, training=False):
    return _run(x, edge_index, batch, W1, b1, W2, b2)
